# speedup vs baseline: 1.6444x; 1.6444x over previous
import functools

import jax
import jax.numpy as jnp
from jax import lax
from jax.experimental import pallas as pl
from jax.experimental.pallas import tpu as pltpu
from jax.experimental.pallas import tpu_sc as plsc

_D = 2304
_B = 8192
_NW = 32
_BPW = _B // _NW
_CHUNK = 16
_NCHUNK = _BPW // _CHUNK


def _make_gather():
  mesh = plsc.VectorSubcoreMesh(core_axis_name="c", subcore_axis_name="s")

  @functools.partial(
      pl.kernel,
      mesh=mesh,
      out_type=jax.ShapeDtypeStruct((_B, _D), jnp.float32),
      scratch_types=[
          pltpu.VMEM((_BPW,), jnp.int32),
          pltpu.VMEM((2, _CHUNK, _D), jnp.float32),
          pltpu.SemaphoreType.DMA,
          pltpu.SemaphoreType.DMA,
          pltpu.SemaphoreType.DMA,
          pltpu.SemaphoreType.DMA,
      ],
  )
  def gather_kernel(idx_hbm, table_hbm, out_hbm, idx_v, rows_v,
                    gsem0, gsem1, ssem0, ssem1):
    wid = lax.axis_index("s") * 2 + lax.axis_index("c")
    base = wid * _BPW
    pltpu.sync_copy(idx_hbm.at[pl.ds(base, _BPW)], idx_v)

    gsems = (gsem0, gsem1)
    ssems = (ssem0, ssem1)

    def fire_gather(c):
      b = c % 2
      return pltpu.async_copy(
          table_hbm.at[idx_v.at[pl.ds(c * _CHUNK, _CHUNK)]],
          rows_v.at[b], gsems[b])

    def fire_scatter(c):
      b = c % 2
      return pltpu.async_copy(
          rows_v.at[b],
          out_hbm.at[pl.ds(base + c * _CHUNK, _CHUNK)], ssems[b])

    gathers = [None, None]
    scatters = [None, None]
    gathers[0] = fire_gather(0)
    for c in range(_NCHUNK):
      b = c % 2
      nb = (c + 1) % 2
      if c + 1 < _NCHUNK:
        if scatters[nb] is not None:
          scatters[nb].wait()
        gathers[nb] = fire_gather(c + 1)
      gathers[b].wait()
      scatters[b] = fire_scatter(c)
    scatters[(_NCHUNK - 2) % 2].wait()
    scatters[(_NCHUNK - 1) % 2].wait()

  return gather_kernel


_gather = _make_gather()


def kernel(token_ids, table):
  flat_ids = token_ids.reshape(-1).astype(jnp.int32)
  out = _gather(flat_ids, table)
  return out.reshape(token_ids.shape + (table.shape[1],))


# speedup vs baseline: 1.6573x; 1.0078x over previous
import functools

import jax
import jax.numpy as jnp
from jax import lax
from jax.experimental import pallas as pl
from jax.experimental.pallas import tpu as pltpu
from jax.experimental.pallas import tpu_sc as plsc

_D = 2304
_B = 8192
_NW = 32
_BPW = _B // _NW
_CHUNK = 16
_NCHUNK = _BPW // _CHUNK
_NBUF = 3


def _make_gather():
  mesh = plsc.VectorSubcoreMesh(core_axis_name="c", subcore_axis_name="s")

  @functools.partial(
      pl.kernel,
      mesh=mesh,
      out_type=jax.ShapeDtypeStruct((_B, _D), jnp.float32),
      scratch_types=[
          pltpu.VMEM((_BPW,), jnp.int32),
          pltpu.VMEM((_NBUF, _CHUNK, _D), jnp.float32),
      ] + [pltpu.SemaphoreType.DMA] * (2 * _NBUF),
  )
  def gather_kernel(idx_hbm, table_hbm, out_hbm, idx_v, rows_v, *sems):
    gsems = sems[:_NBUF]
    ssems = sems[_NBUF:]
    wid = lax.axis_index("s") * 2 + lax.axis_index("c")
    base = wid * _BPW
    pltpu.sync_copy(idx_hbm.at[pl.ds(base, _BPW)], idx_v)

    def fire_gather(c):
      b = c % _NBUF
      return pltpu.async_copy(
          table_hbm.at[idx_v.at[pl.ds(c * _CHUNK, _CHUNK)]],
          rows_v.at[b], gsems[b])

    def fire_scatter(c):
      b = c % _NBUF
      return pltpu.async_copy(
          rows_v.at[b],
          out_hbm.at[pl.ds(base + c * _CHUNK, _CHUNK)], ssems[b])

    gathers = [None] * _NBUF
    scatters = [None] * _NBUF
    for t in range(_NCHUNK + _NBUF - 1):
      if t < _NCHUNK:
        b = t % _NBUF
        if scatters[b] is not None:
          scatters[b].wait()
          scatters[b] = None
        gathers[b] = fire_gather(t)
      d = t - (_NBUF - 1)
      if d >= 0:
        bd = d % _NBUF
        gathers[bd].wait()
        scatters[bd] = fire_scatter(d)
    for s in scatters:
      if s is not None:
        s.wait()

  return gather_kernel


_gather = _make_gather()


def kernel(token_ids, table):
  flat_ids = token_ids.reshape(-1).astype(jnp.int32)
  out = _gather(flat_ids, table)
  return out.reshape(token_ids.shape + (table.shape[1],))


# speedup vs baseline: 1.6607x; 1.0021x over previous
import functools

import jax
import jax.numpy as jnp
from jax import lax
from jax.experimental import pallas as pl
from jax.experimental.pallas import tpu as pltpu
from jax.experimental.pallas import tpu_sc as plsc

_D = 2304
_S = 4
_T = 2048
_NW = 32
_BPW = _S * _T // _NW
_CHUNK = 16
_NCHUNK = _BPW // _CHUNK
_NBUF = 3


def _make_gather():
  mesh = plsc.VectorSubcoreMesh(core_axis_name="c", subcore_axis_name="s")

  @functools.partial(
      pl.kernel,
      mesh=mesh,
      out_type=jax.ShapeDtypeStruct((_S, _T, _D), jnp.float32),
      scratch_types=[
          pltpu.VMEM((_BPW,), jnp.int32),
          pltpu.VMEM((_NBUF, _CHUNK, _D), jnp.float32),
      ] + [pltpu.SemaphoreType.DMA] * (2 * _NBUF),
  )
  def gather_kernel(idx_hbm, table_hbm, out_hbm, idx_v, rows_v, *sems):
    gsems = sems[:_NBUF]
    ssems = sems[_NBUF:]
    wid = lax.axis_index("s") * 2 + lax.axis_index("c")
    base = wid * _BPW
    seq = base // _T
    off = base % _T
    pltpu.sync_copy(idx_hbm.at[seq, pl.ds(off, _BPW)], idx_v)

    def fire_gather(c):
      b = c % _NBUF
      return pltpu.async_copy(
          table_hbm.at[idx_v.at[pl.ds(c * _CHUNK, _CHUNK)]],
          rows_v.at[b], gsems[b])

    def fire_scatter(c):
      b = c % _NBUF
      return pltpu.async_copy(
          rows_v.at[b],
          out_hbm.at[seq, pl.ds(off + c * _CHUNK, _CHUNK)], ssems[b])

    gathers = [None] * _NBUF
    scatters = [None] * _NBUF
    for t in range(_NCHUNK + _NBUF - 1):
      if t < _NCHUNK:
        b = t % _NBUF
        if scatters[b] is not None:
          scatters[b].wait()
          scatters[b] = None
        gathers[b] = fire_gather(t)
      d = t - (_NBUF - 1)
      if d >= 0:
        bd = d % _NBUF
        gathers[bd].wait()
        scatters[bd] = fire_scatter(d)
    for s in scatters:
      if s is not None:
        s.wait()

  return gather_kernel


_gather = _make_gather()


def kernel(token_ids, table):
  return _gather(token_ids.astype(jnp.int32), table)
